# baseline (speedup 1.0000x reference)
import functools

import jax
import jax.numpy as jnp
from jax import lax
from jax.experimental import pallas as pl
from jax.experimental.pallas import tpu as pltpu
from jax.experimental.pallas import tpu_sc as plsc

VOCAB = 100000
D = 64
T = 2048
B = 4
NTOK = B * T
NW = 32
D_PER_W = D // NW
LANES = 16
NC = 2
H0SZ = 50048
H1MAIN = ((VOCAB - H0SZ) // 128) * 128
TAILW = 128
TAIL0 = VOCAB - TAILW
H1SZ = H1MAIN + TAILW


def _sc_build():
    mesh = plsc.VectorSubcoreMesh(core_axis_name="c", subcore_axis_name="s")

    @functools.partial(
        pl.kernel,
        mesh=mesh,
        out_type=jax.ShapeDtypeStruct((B, D, T), jnp.float32),
        scratch_types=[
            pltpu.VMEM((NTOK,), jnp.int32),
            pltpu.VMEM((H0SZ,), jnp.float32),
            pltpu.VMEM((H1SZ,), jnp.float32),
            pltpu.VMEM((D_PER_W, T), jnp.float32),
            pltpu.VMEM((D_PER_W, B, T), jnp.float32),
            pltpu.SemaphoreType.DMA,
            pltpu.SemaphoreType.DMA,
            pltpu.SemaphoreType.DMA,
        ],
        compiler_params=pltpu.CompilerParams(
            use_tc_tiling_on_sc=True, needs_layout_passes=False
        ),
    )
    def k(tok_hbm, tab_hbm, tail_hbm, pos_hbm, out_hbm, idx_v, h0_v, h1_v,
          pos_v, out_v, s0, s1, osem):
        wid = lax.axis_index("s") * NC + lax.axis_index("c")
        d0 = wid * D_PER_W

        def stage_h0(d, di):
            pltpu.async_copy(tab_hbm.at[d].at[pl.ds(0, H0SZ)], h0_v, s0)
            pltpu.async_copy(pos_hbm.at[d], pos_v.at[di], s0)

        def stage_h1(d):
            pltpu.async_copy(
                tab_hbm.at[d].at[pl.ds(H0SZ, H1MAIN)],
                h1_v.at[pl.ds(0, H1MAIN)],
                s1,
            )
            pltpu.async_copy(
                tail_hbm.at[d], h1_v.at[pl.ds(H1MAIN, TAILW)], s1
            )

        def wait_h0(di):
            pltpu.make_async_copy(
                tab_hbm.at[0].at[pl.ds(0, H0SZ)], h0_v, s0
            ).wait()
            pltpu.make_async_copy(pos_hbm.at[0], pos_v.at[di], s0).wait()

        def wait_h1():
            pltpu.make_async_copy(
                tab_hbm.at[0].at[pl.ds(0, H1MAIN)],
                h1_v.at[pl.ds(0, H1MAIN)],
                s1,
            ).wait()
            pltpu.make_async_copy(
                tail_hbm.at[0], h1_v.at[pl.ds(H1MAIN, TAILW)], s1
            ).wait()

        def pass0(di):
            def grp(g, carry):
                t0 = g * LANES
                sl = pl.ds(t0, LANES)
                pos16 = pos_v[di, sl]
                for b in range(B):
                    idx16 = idx_v[pl.ds(b * T + t0, LANES)]
                    loc = jnp.minimum(idx16, H0SZ - 1)
                    vals = plsc.load_gather(h0_v, [loc])
                    out_v[di, b, sl] = vals + pos16
                return carry

            lax.fori_loop(0, T // LANES, grp, 0)

        def pass1(di):
            def grp(g, carry):
                t0 = g * LANES
                sl = pl.ds(t0, LANES)
                pos16 = pos_v[di, sl]
                for b in range(B):
                    idx16 = idx_v[pl.ds(b * T + t0, LANES)]
                    base = idx16 - H0SZ
                    in_tail = idx16 >= H0SZ + H1MAIN
                    loc = base + jnp.where(in_tail, H0SZ + H1MAIN - TAIL0, 0)
                    loc = jnp.maximum(loc, 0)
                    vals = plsc.load_gather(h1_v, [loc])
                    prev = out_v[di, b, sl]
                    keep = idx16 < H0SZ
                    out_v[di, b, sl] = jnp.where(keep, prev, vals + pos16)
                return carry

            lax.fori_loop(0, T // LANES, grp, 0)

        def flush(di, d):
            for b in range(B):
                pltpu.async_copy(out_v.at[di, b], out_hbm.at[b, d], osem)

        pltpu.sync_copy(tok_hbm, idx_v)

        stage_h0(d0, 0)
        stage_h1(d0)
        wait_h0(0)
        pass0(0)
        stage_h0(d0 + 1, 1)
        wait_h1()
        pass1(0)
        stage_h1(d0 + 1)
        flush(0, d0)
        wait_h0(1)
        pass0(1)
        wait_h1()
        pass1(1)
        flush(1, d0 + 1)

        pltpu.make_async_copy(
            out_hbm.at[pl.ds(0, D_PER_W), pl.ds(0, B)], out_v, osem
        ).wait()

    return k


def kernel(tokens, tok_table, pos_table):
    tokens_1d = tokens.reshape(NTOK).astype(jnp.int32)
    table_t = tok_table.T
    tail_t = tok_table[TAIL0:].T
    pos_t = pos_table.T
    out_t = _sc_build()(tokens_1d, table_t, tail_t, pos_t)
    return out_t.transpose(0, 2, 1)


# speedup vs baseline: 1.0439x; 1.0439x over previous
import functools

import jax
import jax.numpy as jnp
from jax import lax
from jax.experimental import pallas as pl
from jax.experimental.pallas import tpu as pltpu
from jax.experimental.pallas import tpu_sc as plsc

VOCAB = 100000
D = 64
T = 2048
B = 4
NTOK = B * T
NW = 32
D_PER_W = D // NW
LANES = 16
NC = 2
H0SZ = 50048
H1MAIN = ((VOCAB - H0SZ) // 128) * 128
TAILW = 128
TAIL0 = VOCAB - TAILW
H1SZ = H1MAIN + TAILW


def _sc_build():
    mesh = plsc.VectorSubcoreMesh(core_axis_name="c", subcore_axis_name="s")

    @functools.partial(
        pl.kernel,
        mesh=mesh,
        out_type=jax.ShapeDtypeStruct((B, D, T), jnp.float32),
        scratch_types=[
            pltpu.VMEM((NTOK,), jnp.int32),
            pltpu.VMEM((H0SZ,), jnp.float32),
            pltpu.VMEM((H1SZ,), jnp.float32),
            pltpu.VMEM((D_PER_W, T), jnp.float32),
            pltpu.VMEM((D_PER_W, B, T), jnp.float32),
            pltpu.SemaphoreType.DMA,
            pltpu.SemaphoreType.DMA,
            pltpu.SemaphoreType.DMA,
        ],
        compiler_params=pltpu.CompilerParams(
            use_tc_tiling_on_sc=True, needs_layout_passes=False
        ),
    )
    def k(tok_hbm, tab_hbm, tail_hbm, pos_hbm, out_hbm, idx_v, h0_v, h1_v,
          pos_v, out_v, s0, s1, osem):
        wid = lax.axis_index("s") * NC + lax.axis_index("c")
        d0 = wid * D_PER_W

        def stage_h0(d, di):
            pltpu.async_copy(tab_hbm.at[d].at[pl.ds(0, H0SZ)], h0_v, s0)
            pltpu.async_copy(pos_hbm.at[d], pos_v.at[di], s0)

        def stage_h1(d):
            pltpu.async_copy(
                tab_hbm.at[d].at[pl.ds(H0SZ, H1MAIN)],
                h1_v.at[pl.ds(0, H1MAIN)],
                s1,
            )
            pltpu.async_copy(
                tail_hbm.at[d], h1_v.at[pl.ds(H1MAIN, TAILW)], s1
            )

        def wait_h0(di):
            pltpu.make_async_copy(
                tab_hbm.at[0].at[pl.ds(0, H0SZ)], h0_v, s0
            ).wait()
            pltpu.make_async_copy(pos_hbm.at[0], pos_v.at[di], s0).wait()

        def wait_h1():
            pltpu.make_async_copy(
                tab_hbm.at[0].at[pl.ds(0, H1MAIN)],
                h1_v.at[pl.ds(0, H1MAIN)],
                s1,
            ).wait()
            pltpu.make_async_copy(
                tail_hbm.at[0], h1_v.at[pl.ds(H1MAIN, TAILW)], s1
            ).wait()

        def pass0(di):
            def grp(g, carry):
                t0 = g * LANES
                sl = pl.ds(t0, LANES)
                pos16 = pos_v[di, sl]
                for b in range(B):
                    idx16 = idx_v[pl.ds(b * T + t0, LANES)]
                    loc = jnp.minimum(idx16, H0SZ - 1)
                    vals = plsc.load_gather(h0_v, [loc])
                    out_v[di, b, sl] = vals + pos16
                return carry

            lax.fori_loop(0, T // LANES, grp, 0)

        def pass1(di):
            def grp(g, carry):
                t0 = g * LANES
                sl = pl.ds(t0, LANES)
                pos16 = pos_v[di, sl]
                for b in range(B):
                    idx16 = idx_v[pl.ds(b * T + t0, LANES)]
                    base = idx16 - H0SZ
                    in_tail = idx16 >= H0SZ + H1MAIN
                    loc = base + jnp.where(in_tail, H0SZ + H1MAIN - TAIL0, 0)
                    loc = jnp.maximum(loc, 0)
                    vals = plsc.load_gather(h1_v, [loc])
                    prev = out_v[di, b, sl]
                    keep = idx16 < H0SZ
                    out_v[di, b, sl] = jnp.where(keep, prev, vals + pos16)
                return carry

            lax.fori_loop(0, T // LANES, grp, 0)

        def flush(di, d):
            for b in range(B):
                pltpu.async_copy(out_v.at[di, b], out_hbm.at[b, d], osem)

        pltpu.sync_copy(tok_hbm, idx_v)

        stage_h0(d0, 0)
        wait_h0(0)
        stage_h1(d0)
        pass0(0)
        wait_h1()
        stage_h0(d0 + 1, 1)
        pass1(0)
        stage_h1(d0 + 1)
        flush(0, d0)
        wait_h0(1)
        pass0(1)
        wait_h1()
        pass1(1)
        flush(1, d0 + 1)

        pltpu.make_async_copy(
            out_hbm.at[pl.ds(0, D_PER_W), pl.ds(0, B)], out_v, osem
        ).wait()

    return k


def kernel(tokens, tok_table, pos_table):
    tokens_1d = tokens.reshape(NTOK).astype(jnp.int32)
    table_t = tok_table.T
    tail_t = tok_table[TAIL0:].T
    pos_t = pos_table.T
    out_t = _sc_build()(tokens_1d, table_t, tail_t, pos_t)
    return out_t.transpose(0, 2, 1)


# speedup vs baseline: 1.3426x; 1.2862x over previous
import functools

import jax
import jax.numpy as jnp
from jax import lax
from jax.experimental import pallas as pl
from jax.experimental.pallas import tpu as pltpu
from jax.experimental.pallas import tpu_sc as plsc

VOCAB = 100000
D = 64
T = 2048
B = 4
NTOK = B * T
NW = 32
D_PER_W = D // NW
LANES = 16
NC = 2


def _sc_embed(tokens_1d, table_t, pos_t):
    mesh = plsc.VectorSubcoreMesh(core_axis_name="c", subcore_axis_name="s")

    @functools.partial(
        pl.kernel,
        mesh=mesh,
        out_type=jax.ShapeDtypeStruct((B, D, T), jnp.float32),
        scratch_types=[
            pltpu.VMEM((NTOK,), jnp.int32),
            pltpu.VMEM((VOCAB,), jnp.float32),
            pltpu.VMEM((T,), jnp.float32),
            pltpu.VMEM((D_PER_W, B, T), jnp.float32),
            pltpu.SemaphoreType.DMA,
            pltpu.SemaphoreType.DMA,
        ],
        compiler_params=pltpu.CompilerParams(
            use_tc_tiling_on_sc=True, needs_layout_passes=False
        ),
    )
    def k(tok_hbm, tab_hbm, pos_hbm, out_hbm, idx_v, row_v, pos_v, out_v,
          sem, osem):
        wid = lax.axis_index("s") * NC + lax.axis_index("c")

        h_row = pltpu.async_copy(tab_hbm.at[wid * D_PER_W], row_v, sem)
        h_pos = pltpu.async_copy(pos_hbm.at[wid * D_PER_W], pos_v, sem)
        pltpu.sync_copy(tok_hbm, idx_v)
        hs = [h_row, h_pos]

        for di in range(D_PER_W):
            d = wid * D_PER_W + di
            for h in hs:
                h.wait()

            def grp(g, carry, di=di):
                t0 = g * LANES
                sl = pl.ds(t0, LANES)
                pos16 = pos_v[sl]
                for b in range(B):
                    idx16 = idx_v[pl.ds(b * T + t0, LANES)]
                    vals = plsc.load_gather(row_v, [idx16])
                    out_v[di, b, sl] = vals + pos16
                return carry

            lax.fori_loop(0, T // LANES, grp, 0)

            if di + 1 < D_PER_W:
                hs = [
                    pltpu.async_copy(tab_hbm.at[d + 1], row_v, sem),
                    pltpu.async_copy(pos_hbm.at[d + 1], pos_v, sem),
                ]

            for b in range(B):
                pltpu.async_copy(out_v.at[di, b], out_hbm.at[b, d], osem)

        pltpu.make_async_copy(
            out_hbm.at[pl.ds(0, D_PER_W), pl.ds(0, B)], out_v, osem
        ).wait()

    return k(tokens_1d, table_t, pos_t)


def kernel(tokens, tok_table, pos_table):
    tokens_1d = tokens.reshape(NTOK).astype(jnp.int32)
    out_t = _sc_embed(tokens_1d, tok_table.T, pos_table.T)
    return out_t.transpose(0, 2, 1)
